# Initial kernel scaffold; baseline (speedup 1.0000x reference)
import jax
import jax.numpy as jnp
from jax.experimental import pallas as pl


def kernel(x, W_hid, b_hid, W_exp, b_exp, rkhs_embeddings):
    raise NotImplementedError("write your pallas kernel here")


# speedup vs baseline: 1.0226x; 1.0226x over previous
import jax
import jax.numpy as jnp
from jax import lax
from jax.experimental import pallas as pl
from jax.experimental.pallas import tpu as pltpu


def _prep_body(emb_ref, wexp_ref, bexp_ref, emb2_ref):
    emb2_ref[:] = lax.dot_general(
        emb_ref[:], wexp_ref[:], (((1,), (1,)), ((), ())),
        preferred_element_type=jnp.float32) + bexp_ref[:]


def _make_route_body(n_tokens, n_experts, topk):
    aux_scale = (float(topk) / n_experts) * 0.5 * (n_experts * n_experts) / n_tokens

    def _route_body(x_ref, whid_ref, bhid_ref, emb2_ref,
                    se_ref, rw_ref, aux_ref, acc_ref):
        enc = lax.dot_general(
            x_ref[:], whid_ref[:], (((1,), (1,)), ((), ())),
            preferred_element_type=jnp.float32) + bhid_ref[:]
        logits = lax.dot_general(
            enc, emb2_ref[:], (((1,), (1,)), ((), ())),
            preferred_element_type=jnp.float32)
        ii = lax.broadcasted_iota(jnp.int32, logits.shape, 1)
        big = jnp.int32(n_experts)
        m1 = jnp.max(logits, axis=1, keepdims=True)
        a1 = jnp.min(jnp.where(logits == m1, ii, big), axis=1, keepdims=True)
        masked = jnp.where(ii == a1, -jnp.inf, logits)
        m2 = jnp.max(masked, axis=1, keepdims=True)
        a2 = jnp.min(jnp.where(masked == m2, ii, big), axis=1, keepdims=True)
        e2 = jnp.exp(m2 - m1)
        w1 = 1.0 / (1.0 + e2)
        w2 = e2 / (1.0 + e2)
        se_ref[:] = jnp.concatenate([a1, a2], axis=1)
        rw_ref[:] = jnp.concatenate([w1, w2], axis=1)
        i = pl.program_id(0)

        @pl.when(i == 0)
        def _():
            acc_ref[0, 0] = 0.0

        acc_ref[0, 0] += jnp.sum(w1 + w2)

        @pl.when(i == pl.num_programs(0) - 1)
        def _():
            aux_ref[:, :] = jnp.full((1, 1), acc_ref[0, 0] * aux_scale,
                                     dtype=jnp.float32)

    return _route_body


def kernel(x, W_hid, b_hid, W_exp, b_exp, rkhs_embeddings):
    b, s, d = x.shape
    rkhs = W_hid.shape[0]
    n_experts, emb = rkhs_embeddings.shape
    topk = 2
    n = b * s
    x2 = x.reshape(n, d)

    emb2 = pl.pallas_call(
        _prep_body,
        out_shape=jax.ShapeDtypeStruct((n_experts, rkhs), jnp.float32),
    )(rkhs_embeddings, W_exp, b_exp.reshape(1, rkhs))

    tb = 1024
    se, rw, aux = pl.pallas_call(
        _make_route_body(n, n_experts, topk),
        grid=(n // tb,),
        in_specs=[pl.BlockSpec((tb, d), lambda i: (i, 0)),
                  pl.BlockSpec((rkhs, d), lambda i: (0, 0)),
                  pl.BlockSpec((1, rkhs), lambda i: (0, 0)),
                  pl.BlockSpec((n_experts, rkhs), lambda i: (0, 0))],
        out_specs=[pl.BlockSpec((tb, topk), lambda i: (i, 0)),
                   pl.BlockSpec((tb, topk), lambda i: (i, 0)),
                   pl.BlockSpec((1, 1), lambda i: (0, 0))],
        out_shape=[jax.ShapeDtypeStruct((n, topk), jnp.int32),
                   jax.ShapeDtypeStruct((n, topk), jnp.float32),
                   jax.ShapeDtypeStruct((1, 1), jnp.float32)],
        scratch_shapes=[pltpu.SMEM((1, 1), jnp.float32)],
    )(x2, W_hid, b_hid.reshape(1, rkhs), emb2)

    return (se.reshape(b, s, topk), rw.reshape(b, s, topk),
            aux.reshape(()))
